# baseline (speedup 1.0000x reference)
import functools

import jax
import jax.numpy as jnp
from jax import lax
from jax.experimental import pallas as pl
from jax.experimental.pallas import tpu as pltpu
from jax.experimental.pallas import tpu_sc as plsc

NC = 2
NS = 16
NW = NC * NS
STREAM = 128
NBUF = 5
NCHUNK = 4


def _make_sc_pool(Bc, S, E):
    b_per_w = Bc // NW
    rows_per_w = b_per_w * S
    ns = rows_per_w // STREAM
    assert ns % NBUF == 0 and b_per_w == STREAM

    mesh = plsc.VectorSubcoreMesh(core_axis_name="c", subcore_axis_name="s")

    @functools.partial(
        pl.kernel,
        out_type=jax.ShapeDtypeStruct((Bc, E), jnp.float32),
        mesh=mesh,
        scratch_types=(
            [pltpu.VMEM((ns, STREAM), jnp.int32),
             pltpu.VMEM((ns, STREAM), jnp.int32)]
            + [pltpu.VMEM((STREAM, E), jnp.float32) for _ in range(NBUF)]
            + [pltpu.VMEM_SHARED((NS * b_per_w, E), jnp.float32)]
            + [pltpu.SemaphoreType.DMA for _ in range(2 * NBUF + 1)]
        ),
    )
    def sc_pool(tokens_hbm, rowmap_hbm, zeros_hbm, table_hbm, out_hbm,
                idx_v, map_v, *rest):
        bufs = list(rest[:NBUF])
        pooled = rest[NBUF]
        gsems = list(rest[NBUF + 1:2 * NBUF + 1])
        ssems = list(rest[2 * NBUF + 1:3 * NBUF + 1])
        zsem = rest[3 * NBUF + 1]
        c = lax.axis_index("c")
        s = lax.axis_index("s")
        w = c * NS + s
        base = s * b_per_w

        zcopy = pltpu.async_copy(
            zeros_hbm, pooled.at[pl.ds(base, b_per_w)], zsem)

        pltpu.sync_copy(tokens_hbm.at[w], idx_v)

        for b in range(NBUF):
            pltpu.async_copy(table_hbm.at[idx_v.at[b]], bufs[b], gsems[b])

        pltpu.sync_copy(rowmap_hbm.at[s], map_v)
        zcopy.wait()

        def step(j, b, first):
            pltpu.make_async_copy(
                table_hbm.at[idx_v.at[j]], bufs[b], gsems[b]).wait()
            pltpu.async_copy(
                bufs[b], pooled.at[map_v.at[j]], ssems[b], add=True)
            if not first:
                bp = (b - 1) % NBUF
                pltpu.make_async_copy(
                    bufs[bp], pooled.at[map_v.at[j]], ssems[bp]).wait()
                nxt = j + NBUF - 1

                @pl.when(nxt < ns)
                def _():
                    pltpu.async_copy(
                        table_hbm.at[idx_v.at[nxt]], bufs[bp], gsems[bp])

        for b in range(NBUF):
            step(b, b, first=(b == 0))

        def chunk(i, carry):
            for b in range(NBUF):
                step(i * NBUF + b, b, first=False)
            return carry

        lax.fori_loop(1, ns // NBUF, chunk, 0)
        last = (ns - 1) % NBUF
        pltpu.make_async_copy(
            bufs[last], pooled.at[map_v.at[0]], ssems[last]).wait()

        pltpu.sync_copy(pooled.at[pl.ds(base, b_per_w)],
                        out_hbm.at[pl.ds(w * b_per_w, b_per_w)])

    return sc_pool


def _mlp_body(x_ref, w1_ref, b1_ref, w2_ref, b2_ref, o_ref, *, inv_s):
    x = x_ref[...] * inv_s
    h = jnp.dot(x, w1_ref[...], preferred_element_type=jnp.float32)
    h = jnp.maximum(h + b1_ref[...], 0.0)
    o = jnp.dot(h, w2_ref[...], preferred_element_type=jnp.float32)
    o_ref[...] = jax.nn.sigmoid(o + b2_ref[...])


def kernel(tokens, emb_table, W1, b1, W2, b2):
    B, S = tokens.shape
    V, E = emb_table.shape
    H = W1.shape[1]
    Bc = B // NCHUNK
    b_per_w = Bc // NW
    rows_per_w = b_per_w * S
    n_streams = rows_per_w // STREAM

    tokens = tokens.astype(jnp.int32)
    t = jnp.arange(rows_per_w, dtype=jnp.int32) // S
    rowmap = (jnp.arange(NS, dtype=jnp.int32)[:, None] * b_per_w + t[None, :]
              ).reshape(NS, n_streams, STREAM)

    sc_pool = _make_sc_pool(Bc, S, E)
    zeros = jnp.zeros((b_per_w, E), jnp.float32)
    b1r = b1.reshape(1, H)
    b2r = b2.reshape(1, 1)

    BT = 512
    mlp = pl.pallas_call(
        functools.partial(_mlp_body, inv_s=1.0 / S),
        grid=(Bc // BT,),
        in_specs=[
            pl.BlockSpec((BT, E), lambda i: (i, 0)),
            pl.BlockSpec((E, H), lambda i: (0, 0)),
            pl.BlockSpec((1, H), lambda i: (0, 0)),
            pl.BlockSpec((H, 1), lambda i: (0, 0)),
            pl.BlockSpec((1, 1), lambda i: (0, 0)),
        ],
        out_specs=pl.BlockSpec((BT, 1), lambda i: (i, 0)),
        out_shape=jax.ShapeDtypeStruct((Bc, 1), jnp.float32),
    )

    outs = []
    for ck in range(NCHUNK):
        tokens_c = lax.slice_in_dim(tokens, ck * Bc, (ck + 1) * Bc, axis=0)
        tokens_r = tokens_c.reshape(NW, n_streams, STREAM)
        pooled = sc_pool(tokens_r, rowmap, zeros, emb_table)
        outs.append(mlp(pooled, W1, b1r, W2, b2r))
    return jnp.concatenate(outs, axis=0)


# speedup vs baseline: 1.0447x; 1.0447x over previous
import functools

import jax
import jax.numpy as jnp
from jax import lax
from jax.experimental import pallas as pl
from jax.experimental.pallas import tpu as pltpu
from jax.experimental.pallas import tpu_sc as plsc

NC = 2
NS = 16
NW = NC * NS
STREAM = 128
NBUF = 5
NCHUNK = 4


def _make_sc_pool(Bc, S, E):
    b_per_w = Bc // NW
    rows_per_w = b_per_w * S
    ns = rows_per_w // STREAM
    assert ns % NBUF == 0 and b_per_w == STREAM

    mesh = plsc.VectorSubcoreMesh(core_axis_name="c", subcore_axis_name="s")

    @functools.partial(
        pl.kernel,
        out_type=jax.ShapeDtypeStruct((Bc, E), jnp.float32),
        mesh=mesh,
        scratch_types=(
            [pltpu.VMEM((ns, STREAM), jnp.int32),
             pltpu.VMEM((ns, STREAM), jnp.int32)]
            + [pltpu.VMEM((STREAM, E), jnp.float32) for _ in range(NBUF)]
            + [pltpu.VMEM_SHARED((NS * b_per_w, E), jnp.float32)]
            + [pltpu.SemaphoreType.DMA for _ in range(2 * NBUF + 1)]
        ),
    )
    def sc_pool(tokens_hbm, rowmap_hbm, zeros_hbm, table_hbm, out_hbm,
                idx_v, map_v, *rest):
        bufs = list(rest[:NBUF])
        pooled = rest[NBUF]
        gsems = list(rest[NBUF + 1:2 * NBUF + 1])
        ssems = list(rest[2 * NBUF + 1:3 * NBUF + 1])
        zsem = rest[3 * NBUF + 1]
        c = lax.axis_index("c")
        s = lax.axis_index("s")
        w = c * NS + s
        base = s * b_per_w

        del zeros_hbm, zsem
        pltpu.sync_copy(tokens_hbm.at[w], idx_v)

        zero = jnp.zeros((16,), jnp.float32)

        def zbody(r, carry):
            for kk in range(E // 16):
                bufs[0][r, pl.ds(kk * 16, 16)] = zero
            return carry

        lax.fori_loop(0, STREAM, zbody, 0)
        pltpu.sync_copy(bufs[0], pooled.at[pl.ds(base, b_per_w)])

        for b in range(NBUF):
            pltpu.async_copy(table_hbm.at[idx_v.at[b]], bufs[b], gsems[b])

        pltpu.sync_copy(rowmap_hbm.at[s], map_v)

        def step(j, b, first):
            pltpu.make_async_copy(
                table_hbm.at[idx_v.at[j]], bufs[b], gsems[b]).wait()
            pltpu.async_copy(
                bufs[b], pooled.at[map_v.at[j]], ssems[b], add=True)
            if not first:
                bp = (b - 1) % NBUF
                pltpu.make_async_copy(
                    bufs[bp], pooled.at[map_v.at[j]], ssems[bp]).wait()
                nxt = j + NBUF - 1

                @pl.when(nxt < ns)
                def _():
                    pltpu.async_copy(
                        table_hbm.at[idx_v.at[nxt]], bufs[bp], gsems[bp])

        for b in range(NBUF):
            step(b, b, first=(b == 0))

        def chunk(i, carry):
            for b in range(NBUF):
                step(i * NBUF + b, b, first=False)
            return carry

        lax.fori_loop(1, ns // NBUF, chunk, 0)
        last = (ns - 1) % NBUF
        pltpu.make_async_copy(
            bufs[last], pooled.at[map_v.at[0]], ssems[last]).wait()

        pltpu.sync_copy(pooled.at[pl.ds(base, b_per_w)],
                        out_hbm.at[pl.ds(w * b_per_w, b_per_w)])

    return sc_pool


def _mlp_body(x_ref, w1_ref, b1_ref, w2_ref, b2_ref, o_ref, *, inv_s):
    x = x_ref[...] * inv_s
    h = jnp.dot(x, w1_ref[...], preferred_element_type=jnp.float32)
    h = jnp.maximum(h + b1_ref[...], 0.0)
    o = jnp.dot(h, w2_ref[...], preferred_element_type=jnp.float32)
    o_ref[...] = jax.nn.sigmoid(o + b2_ref[...])


def kernel(tokens, emb_table, W1, b1, W2, b2):
    B, S = tokens.shape
    V, E = emb_table.shape
    H = W1.shape[1]
    Bc = B // NCHUNK
    b_per_w = Bc // NW
    rows_per_w = b_per_w * S
    n_streams = rows_per_w // STREAM

    tokens = tokens.astype(jnp.int32)
    t = jnp.arange(rows_per_w, dtype=jnp.int32) // S
    rowmap = (jnp.arange(NS, dtype=jnp.int32)[:, None] * b_per_w + t[None, :]
              ).reshape(NS, n_streams, STREAM)

    sc_pool = _make_sc_pool(Bc, S, E)
    zeros = jnp.zeros((b_per_w, E), jnp.float32)
    b1r = b1.reshape(1, H)
    b2r = b2.reshape(1, 1)

    BT = 512
    mlp = pl.pallas_call(
        functools.partial(_mlp_body, inv_s=1.0 / S),
        grid=(Bc // BT,),
        in_specs=[
            pl.BlockSpec((BT, E), lambda i: (i, 0)),
            pl.BlockSpec((E, H), lambda i: (0, 0)),
            pl.BlockSpec((1, H), lambda i: (0, 0)),
            pl.BlockSpec((H, 1), lambda i: (0, 0)),
            pl.BlockSpec((1, 1), lambda i: (0, 0)),
        ],
        out_specs=pl.BlockSpec((BT, 1), lambda i: (i, 0)),
        out_shape=jax.ShapeDtypeStruct((Bc, 1), jnp.float32),
    )

    outs = []
    for ck in range(NCHUNK):
        tokens_c = lax.slice_in_dim(tokens, ck * Bc, (ck + 1) * Bc, axis=0)
        tokens_r = tokens_c.reshape(NW, n_streams, STREAM)
        pooled = sc_pool(tokens_r, rowmap, zeros, emb_table)
        outs.append(mlp(pooled, W1, b1r, W2, b2r))
    return jnp.concatenate(outs, axis=0)


# speedup vs baseline: 1.0502x; 1.0052x over previous
import functools

import jax
import jax.numpy as jnp
from jax import lax
from jax.experimental import pallas as pl
from jax.experimental.pallas import tpu as pltpu
from jax.experimental.pallas import tpu_sc as plsc

NC = 2
NS = 16
NW = NC * NS
STREAM = 128
NBUF = 5
NCHUNK = 4


def _make_sc_pool(Bc, S, E):
    b_per_w = Bc // NW
    rows_per_w = b_per_w * S
    ns = rows_per_w // STREAM
    assert ns % NBUF == 0 and b_per_w == STREAM

    mesh = plsc.VectorSubcoreMesh(core_axis_name="c", subcore_axis_name="s")

    @functools.partial(
        pl.kernel,
        out_type=jax.ShapeDtypeStruct((Bc, E), jnp.float32),
        mesh=mesh,
        scratch_types=(
            [pltpu.VMEM((ns, STREAM), jnp.int32),
             pltpu.VMEM((ns, STREAM), jnp.int32)]
            + [pltpu.VMEM((STREAM, E), jnp.float32) for _ in range(NBUF)]
            + [pltpu.VMEM_SHARED((NS * b_per_w, E), jnp.float32)]
            + [pltpu.SemaphoreType.DMA for _ in range(2 * NBUF + 1)]
        ),
    )
    def sc_pool(tokens_hbm, rowmap_hbm, table_hbm, out_hbm,
                idx_v, map_v, *rest):
        bufs = list(rest[:NBUF])
        pooled = rest[NBUF]
        gsems = list(rest[NBUF + 1:2 * NBUF + 1])
        ssems = list(rest[2 * NBUF + 1:3 * NBUF + 1])
        osem = rest[3 * NBUF + 1]
        c = lax.axis_index("c")
        s = lax.axis_index("s")
        w = c * NS + s
        base = s * b_per_w

        pltpu.sync_copy(tokens_hbm.at[w], idx_v)

        zero = jnp.zeros((16,), jnp.float32)

        def zbody(r, carry):
            for kk in range(E // 16):
                bufs[0][r, pl.ds(kk * 16, 16)] = zero
            return carry

        lax.fori_loop(0, STREAM, zbody, 0)
        pltpu.sync_copy(bufs[0], pooled.at[pl.ds(base, b_per_w)])

        for b in range(NBUF):
            pltpu.async_copy(table_hbm.at[idx_v.at[b]], bufs[b], gsems[b])

        pltpu.sync_copy(rowmap_hbm.at[s], map_v)

        def step(j, b, first):
            pltpu.make_async_copy(
                table_hbm.at[idx_v.at[j]], bufs[b], gsems[b]).wait()
            pltpu.async_copy(
                bufs[b], pooled.at[map_v.at[j]], ssems[b], add=True)
            if not first:
                bp = (b - 1) % NBUF
                pltpu.make_async_copy(
                    bufs[bp], pooled.at[map_v.at[j]], ssems[bp]).wait()
                nxt = j + NBUF - 1

                @pl.when(nxt < ns)
                def _():
                    pltpu.async_copy(
                        table_hbm.at[idx_v.at[nxt]], bufs[bp], gsems[bp])

        for b in range(NBUF):
            step(b, b, first=(b == 0))

        n_iters = ns // NBUF
        cuts = [0]
        for i in range(1, n_iters):
            fin = ((i * NBUF + NBUF - 1) * STREAM) // S
            cuts.append(min(fin - fin % 8, b_per_w))
        cuts.append(b_per_w)
        pieces = [(cuts[i - 1], cuts[i] - cuts[i - 1])
                  for i in range(1, n_iters + 1)]

        def chunk(i, carry):
            for b in range(NBUF):
                step(i * NBUF + b, b, first=False)
            for ii in range(1, n_iters):
                off, ln = pieces[ii - 1]
                if ln <= 0:
                    continue

                @pl.when(i == ii)
                def _(off=off, ln=ln):
                    pltpu.async_copy(
                        pooled.at[pl.ds(base + off, ln)],
                        out_hbm.at[pl.ds(w * b_per_w + off, ln)], osem)
            return carry

        lax.fori_loop(1, n_iters, chunk, 0)
        last = (ns - 1) % NBUF
        pltpu.make_async_copy(
            bufs[last], pooled.at[map_v.at[0]], ssems[last]).wait()

        off, ln = pieces[-1]
        pltpu.sync_copy(pooled.at[pl.ds(base + off, ln)],
                        out_hbm.at[pl.ds(w * b_per_w + off, ln)])
        for off, ln in pieces[:-1]:
            if ln > 0:
                pltpu.make_async_copy(
                    pooled.at[pl.ds(base + off, ln)],
                    out_hbm.at[pl.ds(w * b_per_w + off, ln)], osem).wait()

    return sc_pool


def _mlp_body(x_ref, w1_ref, b1_ref, w2_ref, b2_ref, o_ref, *, inv_s):
    x = x_ref[...] * inv_s
    h = jnp.dot(x, w1_ref[...], preferred_element_type=jnp.float32)
    h = jnp.maximum(h + b1_ref[...], 0.0)
    o = jnp.dot(h, w2_ref[...], preferred_element_type=jnp.float32)
    o_ref[...] = jax.nn.sigmoid(o + b2_ref[...])


def kernel(tokens, emb_table, W1, b1, W2, b2):
    B, S = tokens.shape
    V, E = emb_table.shape
    H = W1.shape[1]
    Bc = B // NCHUNK
    b_per_w = Bc // NW
    rows_per_w = b_per_w * S
    n_streams = rows_per_w // STREAM

    tokens = tokens.astype(jnp.int32)
    t = jnp.arange(rows_per_w, dtype=jnp.int32) // S
    rowmap = (jnp.arange(NS, dtype=jnp.int32)[:, None] * b_per_w + t[None, :]
              ).reshape(NS, n_streams, STREAM)

    sc_pool = _make_sc_pool(Bc, S, E)
    b1r = b1.reshape(1, H)
    b2r = b2.reshape(1, 1)

    BT = 512
    mlp = pl.pallas_call(
        functools.partial(_mlp_body, inv_s=1.0 / S),
        grid=(Bc // BT,),
        in_specs=[
            pl.BlockSpec((BT, E), lambda i: (i, 0)),
            pl.BlockSpec((E, H), lambda i: (0, 0)),
            pl.BlockSpec((1, H), lambda i: (0, 0)),
            pl.BlockSpec((H, 1), lambda i: (0, 0)),
            pl.BlockSpec((1, 1), lambda i: (0, 0)),
        ],
        out_specs=pl.BlockSpec((BT, 1), lambda i: (i, 0)),
        out_shape=jax.ShapeDtypeStruct((Bc, 1), jnp.float32),
    )

    outs = []
    for ck in range(NCHUNK):
        tokens_c = lax.slice_in_dim(tokens, ck * Bc, (ck + 1) * Bc, axis=0)
        tokens_r = tokens_c.reshape(NW, n_streams, STREAM)
        pooled = sc_pool(tokens_r, rowmap, emb_table)
        outs.append(mlp(pooled, W1, b1r, W2, b2r))
    return jnp.concatenate(outs, axis=0)
